# baseline (speedup 1.0000x reference)
import functools

import jax
import jax.numpy as jnp
from jax import lax
from jax.experimental import pallas as pl
from jax.experimental.pallas import tpu as pltpu
from jax.experimental.pallas import tpu_sc as plsc

NODES = 10000
PAD = 10240
EDGES = 320000
D = 128
NC = 2
NS = 16
NW = NC * NS
EPW = EDGES // NW
K = 80
CHUNKS = EPW // K
ZR = 128
RPT = PAD // NS
BR = 1024


SUP = 2
SUPE = SUP * K
CROWS = EDGES // K
RPTILE = 128
CROWS_PAD = NW * RPTILE
NSUP = RPTILE // SUP
TRASH = PAD - 1


def _make_sc_agg():
    mesh = plsc.VectorSubcoreMesh(core_axis_name="c", subcore_axis_name="s")
    out_type = jax.ShapeDtypeStruct((NC, PAD, D), jnp.float32)
    scratch = [
        pltpu.VMEM((SUP, 2, K), jnp.int32),
        pltpu.VMEM((SUP, 2, K), jnp.int32),
        pltpu.VMEM((SUPE, D), jnp.float32),
        pltpu.VMEM((SUPE, D), jnp.float32),
        pltpu.VMEM_SHARED((PAD, D), jnp.float32),
        pltpu.SemaphoreType.DMA,
        pltpu.SemaphoreType.DMA,
        pltpu.SemaphoreType.DMA,
        pltpu.SemaphoreType.DMA,
    ]

    def body(x_hbm, sd_hbm, out_hbm, idx0, idx1, rows0, rows1, acc,
             gsem0, gsem1, ssem0, ssem1):
        c = lax.axis_index("c")
        s = lax.axis_index("s")
        wid = s * NC + c
        zero16 = jnp.zeros((16,), jnp.float32)

        def zero_rows0(i, carry):
            for j in range(D // 16):
                rows0[i, pl.ds(j * 16, 16)] = zero16
            return carry

        lax.fori_loop(0, SUPE, zero_rows0, 0)
        r0 = s * RPT
        for kk in range(RPT // SUPE):
            pltpu.sync_copy(rows0, acc.at[pl.ds(r0 + kk * SUPE, SUPE)])
        plsc.subcore_barrier()

        base = wid * RPTILE

        def drain(rowsb, ssem):
            pltpu.make_async_copy(x_hbm.at[pl.ds(0, SUPE)], rowsb, ssem).wait()

        def sup_iter(t, idxb, rowsb, gsem, ssem, wait_first):
            if wait_first:
                drain(rowsb, ssem)
            pltpu.sync_copy(sd_hbm.at[pl.ds(base + t * SUP, SUP)], idxb)
            gds = [pltpu.async_copy(x_hbm.at[idxb.at[j, 0]],
                                    rowsb.at[pl.ds(j * K, K)], gsem)
                   for j in range(SUP)]
            for g in gds:
                g.wait()
            for j in range(SUP):
                pltpu.async_copy(rowsb.at[pl.ds(j * K, K)],
                                 acc.at[idxb.at[j, 1]], ssem, add=True)

        sup_iter(0, idx0, rows0, gsem0, ssem0, False)
        sup_iter(1, idx1, rows1, gsem1, ssem1, False)

        def pair(i, carry):
            t = 2 + 2 * i
            sup_iter(t, idx0, rows0, gsem0, ssem0, True)
            sup_iter(t + 1, idx1, rows1, gsem1, ssem1, True)
            return carry

        lax.fori_loop(0, (NSUP - 2) // 2, pair, 0)
        drain(rows0, ssem0)
        drain(rows1, ssem1)
        plsc.subcore_barrier()
        pltpu.sync_copy(acc.at[pl.ds(s * RPT, RPT)],
                        out_hbm.at[c, pl.ds(s * RPT, RPT)])

    return functools.partial(
        pl.kernel, mesh=mesh, out_type=out_type,
        scratch_types=tuple(scratch),
        compiler_params=pltpu.CompilerParams(needs_layout_passes=False))(body)


def _make_sc_deg():
    mesh = plsc.VectorSubcoreMesh(core_axis_name="c", subcore_axis_name="s")
    out_type = jax.ShapeDtypeStruct((NW, PAD), jnp.float32)
    scratch = [
        pltpu.VMEM((RPTILE, 2, K), jnp.int32),
        pltpu.VMEM((PAD,), jnp.float32),
    ]

    def body(sd_hbm, degp_hbm, idxall, deg_v):
        c = lax.axis_index("c")
        s = lax.axis_index("s")
        wid = s * NC + c
        zero16 = jnp.zeros((16,), jnp.float32)
        ones16 = jnp.full((16,), 1.0, jnp.float32)

        def zero_deg(i, carry):
            deg_v[pl.ds(i * 16, 16)] = zero16
            return carry

        lax.fori_loop(0, PAD // 16, zero_deg, 0)
        pltpu.sync_copy(sd_hbm.at[pl.ds(wid * RPTILE, RPTILE)], idxall)

        def row(r, carry):
            for q in range(K // 16):
                idx = idxall[r, 1, pl.ds(q * 16, 16)]
                plsc.addupdate_scatter(deg_v, [idx], ones16)
            return carry

        lax.fori_loop(0, RPTILE, row, 0)
        pltpu.sync_copy(deg_v, degp_hbm.at[wid])

    return functools.partial(
        pl.kernel, mesh=mesh, out_type=out_type,
        scratch_types=tuple(scratch),
        compiler_params=pltpu.CompilerParams(needs_layout_passes=False))(body)


def _mm_body(x_ref, w_ref, o_ref):
    o_ref[...] = jnp.dot(x_ref[...], w_ref[...],
                         preferred_element_type=jnp.float32)


def _mm(x, w):
    return pl.pallas_call(
        _mm_body,
        grid=(PAD // BR,),
        in_specs=[pl.BlockSpec((BR, D), lambda i: (i, 0)),
                  pl.BlockSpec((D, D), lambda i: (0, 0))],
        out_specs=pl.BlockSpec((BR, D), lambda i: (i, 0)),
        out_shape=jax.ShapeDtypeStruct((PAD, D), jnp.float32),
    )(x, w)


def _combine_body(h_ref, p_ref, degt_ref, ws_ref, b_ref, wn_ref,
                  o1_ref, o2_ref):
    deg = jnp.sum(degt_ref[...], axis=1, keepdims=True)
    inv = 1.0 / jnp.maximum(deg, 1.0)
    agg = (p_ref[0] + p_ref[1]) * inv
    t = jnp.dot(h_ref[...], ws_ref[...],
                preferred_element_type=jnp.float32) + b_ref[...] + agg
    hr = jnp.maximum(t, 0.0)
    o1_ref[...] = hr
    o2_ref[...] = jnp.dot(hr, wn_ref[...],
                          preferred_element_type=jnp.float32)


def _combine(h, p, degt, ws, b, wn):
    return pl.pallas_call(
        _combine_body,
        grid=(PAD // BR,),
        in_specs=[pl.BlockSpec((BR, D), lambda i: (i, 0)),
                  pl.BlockSpec((NC, BR, D), lambda i: (0, i, 0)),
                  pl.BlockSpec((BR, NW), lambda i: (i, 0)),
                  pl.BlockSpec((D, D), lambda i: (0, 0)),
                  pl.BlockSpec((1, D), lambda i: (0, 0)),
                  pl.BlockSpec((D, D), lambda i: (0, 0))],
        out_specs=[pl.BlockSpec((BR, D), lambda i: (i, 0)),
                   pl.BlockSpec((BR, D), lambda i: (i, 0))],
        out_shape=[jax.ShapeDtypeStruct((PAD, D), jnp.float32),
                   jax.ShapeDtypeStruct((PAD, D), jnp.float32)],
    )(h, p, degt, ws, b, wn)


def _final_body(h_ref, p_ref, degt_ref, ws_ref, b_ref, o_ref):
    deg = jnp.sum(degt_ref[...], axis=1, keepdims=True)
    inv = 1.0 / jnp.maximum(deg, 1.0)
    agg = (p_ref[0] + p_ref[1]) * inv
    o_ref[...] = jnp.dot(h_ref[...], ws_ref[...],
                         preferred_element_type=jnp.float32) + b_ref[...] + agg


def _final(h, p, degt, ws, b):
    return pl.pallas_call(
        _final_body,
        grid=(PAD // BR,),
        in_specs=[pl.BlockSpec((BR, D), lambda i: (i, 0)),
                  pl.BlockSpec((NC, BR, D), lambda i: (0, i, 0)),
                  pl.BlockSpec((BR, NW), lambda i: (i, 0)),
                  pl.BlockSpec((D, D), lambda i: (0, 0)),
                  pl.BlockSpec((1, D), lambda i: (0, 0))],
        out_specs=pl.BlockSpec((BR, D), lambda i: (i, 0)),
        out_shape=jax.ShapeDtypeStruct((PAD, D), jnp.float32),
    )(h, p, degt, ws, b)


def kernel(h, edge_index, W_self0, W_neigh0, b0, W_self1, W_neigh1, b1,
           W_self2, W_neigh2, b2):
    sd = jnp.stack([edge_index[0].reshape(CROWS, K),
                    edge_index[1].reshape(CROWS, K)], axis=1)
    sd = jnp.concatenate(
        [sd, jnp.full((CROWS_PAD - CROWS, 2, K), TRASH, jnp.int32)], axis=0)
    h_pad = jnp.pad(h, ((0, PAD - NODES), (0, 0)))
    b0r = b0.reshape(1, D)
    b1r = b1.reshape(1, D)
    b2r = b2.reshape(1, D)

    sc_agg = _make_sc_agg()
    sc_deg = _make_sc_deg()

    degp = sc_deg(sd)
    hn0 = _mm(h_pad, W_neigh0)
    p0 = sc_agg(hn0, sd)
    degt = degp.T
    h1, hn1 = _combine(h_pad, p0, degt, W_self0, b0r, W_neigh1)
    p1 = sc_agg(hn1, sd)
    h2, hn2 = _combine(h1, p1, degt, W_self1, b1r, W_neigh2)
    p2 = sc_agg(hn2, sd)
    out = _final(h2, p2, degt, W_self2, b2r)
    return out[:NODES]


# speedup vs baseline: 2.8889x; 2.8889x over previous
import functools

import jax
import jax.numpy as jnp
from jax import lax
from jax.experimental import pallas as pl
from jax.experimental.pallas import tpu as pltpu
from jax.experimental.pallas import tpu_sc as plsc

NODES = 10000
PAD = 10240
EDGES = 320000
D = 128
NC = 2
NS = 16
NW = NC * NS
EPW = EDGES // NW
K = 80
CHUNKS = EPW // K
ZR = 128
RPT = PAD // NS
BR = 1024


SUP = 2
SUPE = SUP * K
CROWS = EDGES // K
RPTILE = 126
CROWS_PAD = NW * RPTILE
NSUP = RPTILE // SUP


def _make_sc_agg():
    mesh = plsc.VectorSubcoreMesh(core_axis_name="c", subcore_axis_name="s")
    out_type = jax.ShapeDtypeStruct((NC, PAD, D), jnp.float32)
    scratch = [
        pltpu.VMEM((SUP, 2, K), jnp.int32),
        pltpu.VMEM((SUP, 2, K), jnp.int32),
        pltpu.VMEM((SUPE, D), jnp.float32),
        pltpu.VMEM((SUPE, D), jnp.float32),
        pltpu.VMEM_SHARED((PAD, D), jnp.float32),
        pltpu.SemaphoreType.DMA,
        pltpu.SemaphoreType.DMA,
        pltpu.SemaphoreType.DMA,
        pltpu.SemaphoreType.DMA,
    ]

    def body(x_hbm, sd_hbm, out_hbm, idx0, idx1, rows0, rows1, acc,
             gsem0, gsem1, ssem0, ssem1):
        c = lax.axis_index("c")
        s = lax.axis_index("s")
        wid = s * NC + c
        zero16 = jnp.zeros((16,), jnp.float32)

        def zero_rows0(i, carry):
            for j in range(D // 16):
                rows0[i, pl.ds(j * 16, 16)] = zero16
            return carry

        lax.fori_loop(0, SUPE, zero_rows0, 0)
        r0 = s * RPT
        for kk in range(RPT // SUPE):
            pltpu.sync_copy(rows0, acc.at[pl.ds(r0 + kk * SUPE, SUPE)])
        plsc.subcore_barrier()

        base = wid * RPTILE

        def drain(rowsb, ssem):
            pltpu.make_async_copy(x_hbm.at[pl.ds(0, SUPE)], rowsb, ssem).wait()

        def sup_iter(t, idxb, rowsb, gsem, ssem, wait_first):
            if wait_first:
                drain(rowsb, ssem)
            pltpu.sync_copy(sd_hbm.at[pl.ds(base + t * SUP, SUP)], idxb)
            gds = [pltpu.async_copy(x_hbm.at[idxb.at[j, 0]],
                                    rowsb.at[pl.ds(j * K, K)], gsem)
                   for j in range(SUP)]
            for g in gds:
                g.wait()
            for j in range(SUP):
                pltpu.async_copy(rowsb.at[pl.ds(j * K, K)],
                                 acc.at[idxb.at[j, 1]], ssem, add=True)

        sup_iter(0, idx0, rows0, gsem0, ssem0, False)
        sup_iter(1, idx1, rows1, gsem1, ssem1, False)

        def pair(i, carry):
            t = 2 + 2 * i
            sup_iter(t, idx0, rows0, gsem0, ssem0, True)
            sup_iter(t + 1, idx1, rows1, gsem1, ssem1, True)
            return carry

        lax.fori_loop(0, (NSUP - 3) // 2, pair, 0)
        sup_iter(NSUP - 1, idx0, rows0, gsem0, ssem0, True)
        drain(rows0, ssem0)
        drain(rows1, ssem1)
        plsc.subcore_barrier()
        pltpu.sync_copy(acc.at[pl.ds(s * RPT, RPT)],
                        out_hbm.at[c, pl.ds(s * RPT, RPT)])

    return functools.partial(
        pl.kernel, mesh=mesh, out_type=out_type,
        scratch_types=tuple(scratch),
        compiler_params=pltpu.CompilerParams(needs_layout_passes=False))(body)


def _make_sc_deg():
    mesh = plsc.VectorSubcoreMesh(core_axis_name="c", subcore_axis_name="s")
    out_type = jax.ShapeDtypeStruct((NW, PAD), jnp.float32)
    scratch = [
        pltpu.VMEM((RPTILE, 2, K), jnp.int32),
        pltpu.VMEM((PAD,), jnp.float32),
    ]

    def body(sd_hbm, degp_hbm, idxall, deg_v):
        c = lax.axis_index("c")
        s = lax.axis_index("s")
        wid = s * NC + c
        zero16 = jnp.zeros((16,), jnp.float32)
        ones16 = jnp.full((16,), 1.0, jnp.float32)

        def zero_deg(i, carry):
            deg_v[pl.ds(i * 16, 16)] = zero16
            return carry

        lax.fori_loop(0, PAD // 16, zero_deg, 0)
        pltpu.sync_copy(sd_hbm.at[pl.ds(wid * RPTILE, RPTILE)], idxall)

        def row(r, carry):
            for q in range(K // 16):
                idx = idxall[r, 1, pl.ds(q * 16, 16)]
                plsc.addupdate_scatter(deg_v, [idx], ones16)
            return carry

        lax.fori_loop(0, RPTILE, row, 0)
        pltpu.sync_copy(deg_v, degp_hbm.at[wid])

    return functools.partial(
        pl.kernel, mesh=mesh, out_type=out_type,
        scratch_types=tuple(scratch),
        compiler_params=pltpu.CompilerParams(needs_layout_passes=False))(body)


def _mm_body(x_ref, w_ref, o_ref):
    o_ref[...] = jnp.dot(x_ref[...], w_ref[...],
                         preferred_element_type=jnp.float32)


def _mm(x, w):
    return pl.pallas_call(
        _mm_body,
        grid=(PAD // BR,),
        in_specs=[pl.BlockSpec((BR, D), lambda i: (i, 0)),
                  pl.BlockSpec((D, D), lambda i: (0, 0))],
        out_specs=pl.BlockSpec((BR, D), lambda i: (i, 0)),
        out_shape=jax.ShapeDtypeStruct((PAD, D), jnp.float32),
    )(x, w)


def _combine_body(h_ref, p_ref, degt_ref, ws_ref, b_ref, wn_ref,
                  o1_ref, o2_ref):
    deg = jnp.sum(degt_ref[...], axis=1, keepdims=True)
    inv = 1.0 / jnp.maximum(deg, 1.0)
    agg = (p_ref[0] + p_ref[1]) * inv
    t = jnp.dot(h_ref[...], ws_ref[...],
                preferred_element_type=jnp.float32) + b_ref[...] + agg
    hr = jnp.maximum(t, 0.0)
    o1_ref[...] = hr
    o2_ref[...] = jnp.dot(hr, wn_ref[...],
                          preferred_element_type=jnp.float32)


def _combine(h, p, degt, ws, b, wn):
    return pl.pallas_call(
        _combine_body,
        grid=(PAD // BR,),
        in_specs=[pl.BlockSpec((BR, D), lambda i: (i, 0)),
                  pl.BlockSpec((NC, BR, D), lambda i: (0, i, 0)),
                  pl.BlockSpec((BR, NW), lambda i: (i, 0)),
                  pl.BlockSpec((D, D), lambda i: (0, 0)),
                  pl.BlockSpec((1, D), lambda i: (0, 0)),
                  pl.BlockSpec((D, D), lambda i: (0, 0))],
        out_specs=[pl.BlockSpec((BR, D), lambda i: (i, 0)),
                   pl.BlockSpec((BR, D), lambda i: (i, 0))],
        out_shape=[jax.ShapeDtypeStruct((PAD, D), jnp.float32),
                   jax.ShapeDtypeStruct((PAD, D), jnp.float32)],
    )(h, p, degt, ws, b, wn)


def _final_body(h_ref, p_ref, degt_ref, ws_ref, b_ref, o_ref):
    deg = jnp.sum(degt_ref[...], axis=1, keepdims=True)
    inv = 1.0 / jnp.maximum(deg, 1.0)
    agg = (p_ref[0] + p_ref[1]) * inv
    o_ref[...] = jnp.dot(h_ref[...], ws_ref[...],
                         preferred_element_type=jnp.float32) + b_ref[...] + agg


def _final(h, p, degt, ws, b):
    return pl.pallas_call(
        _final_body,
        grid=(PAD // BR,),
        in_specs=[pl.BlockSpec((BR, D), lambda i: (i, 0)),
                  pl.BlockSpec((NC, BR, D), lambda i: (0, i, 0)),
                  pl.BlockSpec((BR, NW), lambda i: (i, 0)),
                  pl.BlockSpec((D, D), lambda i: (0, 0)),
                  pl.BlockSpec((1, D), lambda i: (0, 0))],
        out_specs=pl.BlockSpec((BR, D), lambda i: (i, 0)),
        out_shape=jax.ShapeDtypeStruct((PAD, D), jnp.float32),
    )(h, p, degt, ws, b)


def kernel(h, edge_index, W_self0, W_neigh0, b0, W_self1, W_neigh1, b1,
           W_self2, W_neigh2, b2):
    sd = jnp.stack([edge_index[0].reshape(CROWS, K),
                    edge_index[1].reshape(CROWS, K)], axis=1)
    trash = NODES + (jnp.arange((CROWS_PAD - CROWS) * K, dtype=jnp.int32)
                     % (PAD - NODES))
    trash = trash.reshape(CROWS_PAD - CROWS, 1, K)
    sd = jnp.concatenate(
        [sd, jnp.broadcast_to(trash, (CROWS_PAD - CROWS, 2, K))], axis=0)
    h_pad = jnp.pad(h, ((0, PAD - NODES), (0, 0)))
    b0r = b0.reshape(1, D)
    b1r = b1.reshape(1, D)
    b2r = b2.reshape(1, D)

    sc_agg = _make_sc_agg()
    sc_deg = _make_sc_deg()

    degp = sc_deg(sd)
    hn0 = _mm(h_pad, W_neigh0)
    p0 = sc_agg(hn0, sd)
    degt = degp.T
    h1, hn1 = _combine(h_pad, p0, degt, W_self0, b0r, W_neigh1)
    p1 = sc_agg(hn1, sd)
    h2, hn2 = _combine(h1, p1, degt, W_self1, b1r, W_neigh2)
    p2 = sc_agg(hn2, sd)
    out = _final(h2, p2, degt, W_self2, b2r)
    return out[:NODES]
